# Initial kernel scaffold; baseline (speedup 1.0000x reference)
import jax
import jax.numpy as jnp
from jax.experimental import pallas as pl


def kernel(gt_labels, gt_boxes, anchors_cxcywh):
    raise NotImplementedError("write your pallas kernel here")


# speedup vs baseline: 5.3598x; 5.3598x over previous
import functools

import jax
import jax.numpy as jnp
from jax.experimental import pallas as pl
from jax.experimental.pallas import tpu as pltpu

_BACKGROUND_ID = 0
_EPS = 1e-06
_ROWS = 192
_LANES = 128
_APAD = _ROWS * _LANES


def _body(gt_ref, lbl_ref, anc_ref, lbl_out, pos_out, enc_out, *, num_gt, num_anchors):
    acx = anc_ref[0]
    acy = anc_ref[1]
    aw = anc_ref[2]
    ah = anc_ref[3]
    ax1 = acx - aw / 2
    ay1 = acy - ah / 2
    ax2 = acx + aw / 2
    ay2 = acy + ah / 2
    area_a = (ax2 - ax1) * (ay2 - ay1)

    row_i = jax.lax.broadcasted_iota(jnp.int32, (_ROWS, _LANES), 0)
    lane_i = jax.lax.broadcasted_iota(jnp.int32, (_ROWS, _LANES), 1)
    flat = row_i * _LANES + lane_i

    def gt_scalar(c, n):
        return gt_ref[0, 0, c * _LANES + n]

    def step(n, carry):
        best_iou, best_idx, ovr_n, ovr = carry
        gx1 = gt_scalar(0, n)
        gy1 = gt_scalar(1, n)
        gx2 = gt_scalar(2, n)
        gy2 = gt_scalar(3, n)
        ltx = jnp.maximum(ax1, gx1)
        lty = jnp.maximum(ay1, gy1)
        rbx = jnp.minimum(ax2, gx2)
        rby = jnp.minimum(ay2, gy2)
        wx = jnp.clip(rbx - ltx, 0.0, None)
        wy = jnp.clip(rby - lty, 0.0, None)
        inter = wx * wy
        area_b = (gx2 - gx1) * (gy2 - gy1)
        union = area_a + area_b - inter
        iou = inter / jnp.maximum(union, 1e-12)
        better = iou > best_iou
        best_iou = jnp.where(better, iou, best_iou)
        best_idx = jnp.where(better, n, best_idx)
        m = jnp.max(iou)
        gidx = jnp.min(jnp.where(iou == m, flat, _APAD))
        hit = flat == gidx
        ovr_n = jnp.where(hit, n, ovr_n)
        ovr = jnp.where(hit, 1, ovr)
        return best_iou, best_idx, ovr_n, ovr

    zero_i = jnp.zeros((_ROWS, _LANES), jnp.int32)
    init = (
        jnp.full((_ROWS, _LANES), -1.0, jnp.float32),
        zero_i,
        zero_i,
        zero_i,
    )
    best_iou, best_idx, ovr_n, ovr_i = jax.lax.fori_loop(0, num_gt, step, init)

    ovr = ovr_i != 0
    pos = jnp.logical_or(best_iou > 0.5, ovr)
    fidx = jnp.where(ovr, ovr_n, best_idx)

    def gather(n, carry):
        lblv, mcx, mcy, mw, mh = carry
        sel = fidx == n
        gx1 = gt_scalar(0, n)
        gy1 = gt_scalar(1, n)
        gx2 = gt_scalar(2, n)
        gy2 = gt_scalar(3, n)
        cx = (gx1 + gx2) / 2
        cy = (gy1 + gy2) / 2
        w = gx2 - gx1
        h = gy2 - gy1
        lab = lbl_ref[0, 0, n]
        lblv = jnp.where(sel, lab, lblv)
        mcx = jnp.where(sel, cx, mcx)
        mcy = jnp.where(sel, cy, mcy)
        mw = jnp.where(sel, w, mw)
        mh = jnp.where(sel, h, mh)
        return lblv, mcx, mcy, mw, mh

    zero_f = jnp.zeros((_ROWS, _LANES), jnp.float32)
    lblv, mcx, mcy, mw, mh = jax.lax.fori_loop(
        0, num_gt, gather, (zero_i, zero_f, zero_f, zero_f, zero_f)
    )

    lbl_out[0] = jnp.where(pos, lblv, _BACKGROUND_ID)
    pos_out[0] = pos.astype(jnp.int32)
    enc_out[0, 0] = (mcx - acx) / aw
    enc_out[0, 1] = (mcy - acy) / ah
    enc_out[0, 2] = jnp.log((mw + _EPS) / (aw + _EPS))
    enc_out[0, 3] = jnp.log((mh + _EPS) / (ah + _EPS))


def kernel(gt_labels, gt_boxes, anchors_cxcywh):
    B, N = gt_boxes.shape[0], gt_boxes.shape[1]
    A = anchors_cxcywh.shape[0]

    gt_t = jnp.transpose(gt_boxes, (0, 2, 1))
    gt_pad = jnp.pad(gt_t, ((0, 0), (0, 0), (0, _LANES - N)))
    gt_flat = gt_pad.reshape(B, 1, 4 * _LANES)
    lbl_pad = jnp.pad(gt_labels, ((0, 0), (0, _LANES - N))).reshape(B, 1, _LANES)

    npad = _APAD - A
    pad_anc = jnp.tile(
        jnp.array([[-10.0, -10.0, 1e-3, 1e-3]], jnp.float32), (npad, 1)
    )
    anc_pad = jnp.concatenate([anchors_cxcywh, pad_anc], axis=0)
    anc_t = jnp.transpose(anc_pad, (1, 0)).reshape(4, _ROWS, _LANES)

    body = functools.partial(_body, num_gt=N, num_anchors=A)
    lbl_full, pos_full, enc_full = pl.pallas_call(
        body,
        grid=(B,),
        in_specs=[
            pl.BlockSpec((1, 1, 4 * _LANES), lambda b: (b, 0, 0), memory_space=pltpu.SMEM),
            pl.BlockSpec((1, 1, _LANES), lambda b: (b, 0, 0), memory_space=pltpu.SMEM),
            pl.BlockSpec((4, _ROWS, _LANES), lambda b: (0, 0, 0)),
        ],
        out_specs=[
            pl.BlockSpec((1, _ROWS, _LANES), lambda b: (b, 0, 0)),
            pl.BlockSpec((1, _ROWS, _LANES), lambda b: (b, 0, 0)),
            pl.BlockSpec((1, 4, _ROWS, _LANES), lambda b: (b, 0, 0, 0)),
        ],
        out_shape=[
            jax.ShapeDtypeStruct((B, _ROWS, _LANES), jnp.int32),
            jax.ShapeDtypeStruct((B, _ROWS, _LANES), jnp.int32),
            jax.ShapeDtypeStruct((B, 4, _ROWS, _LANES), jnp.float32),
        ],
    )(gt_flat, lbl_pad, anc_t)

    encoded_labels = lbl_full.reshape(B, _APAD)[:, :A]
    pos_mask = pos_full.reshape(B, _APAD)[:, :A] != 0
    encoded = jnp.transpose(enc_full.reshape(B, 4, _APAD), (0, 2, 1))[:, :A, :]
    return encoded_labels, encoded, pos_mask
